# Initial kernel scaffold; baseline (speedup 1.0000x reference)
import jax
import jax.numpy as jnp
from jax.experimental import pallas as pl


def kernel(d, L_values, edge_index, matrix_values, mask, residual):
    raise NotImplementedError("write your pallas kernel here")


# speedup vs baseline: 147.5948x; 147.5948x over previous
import functools

import jax
import jax.numpy as jnp
from jax import lax
from jax.experimental import pallas as pl
from jax.experimental.pallas import tpu as pltpu
from jax.experimental.pallas import tpu_sc as plsc

_NC = 2
_NS = 16
_LANES = 16


def _sc_spmv(src, dst, matrix_values, d, n_pad, chunk):
  n = d.shape[0]
  e = matrix_values.shape[0]
  n_tiles = _NC * _NS
  ept = e // n_tiles
  nch = ept // chunk
  sl = n_pad // _NS

  mesh = plsc.VectorSubcoreMesh(core_axis_name="c", subcore_axis_name="s")

  @functools.partial(
      pl.kernel,
      mesh=mesh,
      out_type=jax.ShapeDtypeStruct((_NC, n_pad), jnp.float32),
      compiler_params=pltpu.CompilerParams(needs_layout_passes=False),
      scratch_types=[
          pltpu.VMEM((n,), jnp.float32),
          pltpu.VMEM((chunk,), jnp.int32),
          pltpu.VMEM((chunk,), jnp.int32),
          pltpu.VMEM((chunk,), jnp.float32),
          pltpu.VMEM((chunk,), jnp.float32),
          pltpu.VMEM((n_pad // _NS,), jnp.float32),
          pltpu.VMEM_SHARED((n_pad,), jnp.float32),
      ],
  )
  def spmv(src_hbm, dst_hbm, mv_hbm, d_hbm, out_hbm,
           d_t, src_v, dst_v, mv_v, msg_v, zv, ad_s):
    c = lax.axis_index("c")
    s = lax.axis_index("s")
    wid = c * _NS + s

    pltpu.sync_copy(d_hbm, d_t)

    def zbody(j, carry):
      zv[pl.ds(j * _LANES, _LANES)] = jnp.zeros((_LANES,), jnp.float32)
      return carry

    lax.fori_loop(0, sl // _LANES, zbody, 0)
    pltpu.sync_copy(zv, ad_s.at[pl.ds(s * sl, sl)])
    plsc.subcore_barrier()

    def chunk_body(i, carry):
      base = wid * ept + i * chunk
      pltpu.sync_copy(src_hbm.at[pl.ds(base, chunk)], src_v)
      pltpu.sync_copy(dst_hbm.at[pl.ds(base, chunk)], dst_v)
      pltpu.sync_copy(mv_hbm.at[pl.ds(base, chunk)], mv_v)

      def grp(j, c2):
        o = j * _LANES
        idx = src_v[pl.ds(o, _LANES)]
        vals = plsc.load_gather(d_t, [idx])
        msg_v[pl.ds(o, _LANES)] = vals * mv_v[pl.ds(o, _LANES)]
        return c2

      lax.fori_loop(0, chunk // _LANES, grp, 0)
      pltpu.sync_copy(msg_v, ad_s.at[dst_v], add=True)
      return carry

    lax.fori_loop(0, nch, chunk_body, 0)

    plsc.subcore_barrier()
    off = s * sl
    pltpu.sync_copy(ad_s.at[pl.ds(off, sl)], out_hbm.at[c, pl.ds(off, sl)])

  return spmv(src, dst, matrix_values, d)


def _tc_reduce(partial3, resp, maskp):
  def body(p_ref, r_ref, m_ref, o_ref):
    ad = p_ref[0] + p_ref[1]
    o_ref[0, 0] = jnp.sum(jnp.abs(ad * m_ref[...] - r_ref[...]))

  return pl.pallas_call(
      body,
      out_shape=jax.ShapeDtypeStruct((1, 1), jnp.float32),
      out_specs=pl.BlockSpec(memory_space=pltpu.SMEM),
  )(partial3, resp, maskp)


def kernel(d, L_values, edge_index, matrix_values, mask, residual):
  n = d.shape[0]
  n_pad = 102400
  partial = _sc_spmv(edge_index[0], edge_index[1], matrix_values, d,
                     n_pad, 2000)
  nr = n_pad // 128
  maskp = jnp.pad(mask.astype(jnp.float32), (0, n_pad - n)).reshape(nr, 128)
  resp = jnp.pad(residual, (0, n_pad - n)).reshape(nr, 128)
  tot = _tc_reduce(partial.reshape(_NC, nr, 128), resp, maskp)
  return tot[0, 0] / n


# speedup vs baseline: 312.7915x; 2.1193x over previous
import functools

import jax
import jax.numpy as jnp
from jax import lax
from jax.experimental import pallas as pl
from jax.experimental.pallas import tpu as pltpu
from jax.experimental.pallas import tpu_sc as plsc

_NC = 2
_NS = 16
_LANES = 16
_NBUF = 3


def _sc_spmv(src, dst, matrix_values, d, n_pad, chunk):
  n = d.shape[0]
  e = matrix_values.shape[0]
  n_tiles = _NC * _NS
  ept = e // n_tiles
  nch = ept // chunk
  sl = n_pad // _NS
  assert ept % chunk == 0 and chunk % _LANES == 0 and sl % 8 == 0

  mesh = plsc.VectorSubcoreMesh(core_axis_name="c", subcore_axis_name="s")

  vbufs = []
  for _ in range(_NBUF):
    vbufs += [
        pltpu.VMEM((chunk,), jnp.int32),
        pltpu.VMEM((chunk,), jnp.int32),
        pltpu.VMEM((chunk,), jnp.float32),
        pltpu.VMEM((chunk,), jnp.float32),
    ]

  @functools.partial(
      pl.kernel,
      mesh=mesh,
      out_type=jax.ShapeDtypeStruct((_NC, n_pad), jnp.float32),
      compiler_params=pltpu.CompilerParams(needs_layout_passes=False),
      scratch_types=(
          [pltpu.VMEM((n,), jnp.float32)]
          + vbufs
          + [pltpu.VMEM_SHARED((n_pad,), jnp.float32)]
          + [pltpu.SemaphoreType.DMA] * (2 * _NBUF)
      ),
  )
  def spmv(src_hbm, dst_hbm, mv_hbm, d_hbm, out_hbm, d_t, *rest):
    bufs = [rest[4 * b:4 * b + 4] for b in range(_NBUF)]
    ad_s = rest[4 * _NBUF]
    lsem = rest[4 * _NBUF + 1:4 * _NBUF + 1 + _NBUF]
    ssem = rest[4 * _NBUF + 1 + _NBUF:]
    c = lax.axis_index("c")
    s = lax.axis_index("s")
    wid = c * _NS + s
    tbase = wid * ept

    d_copy = pltpu.async_copy(d_hbm, d_t, lsem[0])

    zv = bufs[0][3]

    def zbody(j, carry):
      zv[pl.ds(j * _LANES, _LANES)] = jnp.zeros((_LANES,), jnp.float32)
      return carry

    lax.fori_loop(0, chunk // _LANES, zbody, 0)
    for q in range(sl // chunk):
      pltpu.sync_copy(zv, ad_s.at[pl.ds(s * sl + q * chunk, chunk)])
    rem = sl % chunk
    if rem:
      pltpu.sync_copy(zv.at[pl.ds(0, rem)],
                      ad_s.at[pl.ds(s * sl + (sl // chunk) * chunk, rem)])
    d_copy.wait()
    plsc.subcore_barrier()

    def issue_loads(i):
      b = i % _NBUF
      base = tbase + i * chunk
      return [
          pltpu.async_copy(src_hbm.at[pl.ds(base, chunk)], bufs[b][0], lsem[b]),
          pltpu.async_copy(dst_hbm.at[pl.ds(base, chunk)], bufs[b][1], lsem[b]),
          pltpu.async_copy(mv_hbm.at[pl.ds(base, chunk)], bufs[b][2], lsem[b]),
      ]

    pending_loads = {0: issue_loads(0)}
    pending_scat = {}
    for i in range(nch):
      b = i % _NBUF
      j = i - (_NBUF - 1)
      if j in pending_scat:
        pending_scat.pop(j).wait()
      if i + 1 < nch:
        pending_loads[i + 1] = issue_loads(i + 1)
      for h in pending_loads.pop(i):
        h.wait()
      src_v, dst_v, mv_v, msg_v = bufs[b]

      def grp(jj, c2, src_v=src_v, mv_v=mv_v, msg_v=msg_v):
        o = jj * _LANES
        idx = src_v[pl.ds(o, _LANES)]
        vals = plsc.load_gather(d_t, [idx])
        msg_v[pl.ds(o, _LANES)] = vals * mv_v[pl.ds(o, _LANES)]
        return c2

      lax.fori_loop(0, chunk // _LANES, grp, 0)
      pending_scat[i] = pltpu.async_copy(
          msg_v, ad_s.at[dst_v], ssem[b], add=True)
    for i in sorted(pending_scat):
      pending_scat[i].wait()

    plsc.subcore_barrier()
    off = s * sl
    pltpu.sync_copy(ad_s.at[pl.ds(off, sl)], out_hbm.at[c, pl.ds(off, sl)])

  return spmv(src, dst, matrix_values, d)


def _tc_reduce(partial3, resp, maskp):
  def body(p_ref, r_ref, m_ref, o_ref):
    ad = p_ref[0] + p_ref[1]
    o_ref[0, 0] = jnp.sum(jnp.abs(ad * m_ref[...] - r_ref[...]))

  return pl.pallas_call(
      body,
      out_shape=jax.ShapeDtypeStruct((1, 1), jnp.float32),
      out_specs=pl.BlockSpec(memory_space=pltpu.SMEM),
  )(partial3, resp, maskp)


def kernel(d, L_values, edge_index, matrix_values, mask, residual):
  n = d.shape[0]
  n_pad = 102400
  partial = _sc_spmv(edge_index[0], edge_index[1], matrix_values, d,
                     n_pad, 2000)
  nr = n_pad // 128
  maskp = jnp.pad(mask.astype(jnp.float32), (0, n_pad - n)).reshape(nr, 128)
  resp = jnp.pad(residual, (0, n_pad - n)).reshape(nr, 128)
  tot = _tc_reduce(partial.reshape(_NC, nr, 128), resp, maskp)
  return tot[0, 0] / n


# speedup vs baseline: 322.0196x; 1.0295x over previous
import functools

import jax
import jax.numpy as jnp
from jax import lax
from jax.experimental import pallas as pl
from jax.experimental.pallas import tpu as pltpu
from jax.experimental.pallas import tpu_sc as plsc

_NC = 2
_NS = 16
_LANES = 16
_NBUF = 3


def _sc_spmv(ei_flat, matrix_values, d, n_pad, chunk):
  n = d.shape[0]
  e = matrix_values.shape[0]
  n_tiles = _NC * _NS
  ept = e // n_tiles
  nch = ept // chunk
  sl = n_pad // _NS
  assert ept % chunk == 0 and chunk % _LANES == 0 and sl % 8 == 0

  mesh = plsc.VectorSubcoreMesh(core_axis_name="c", subcore_axis_name="s")

  vbufs = []
  for _ in range(_NBUF):
    vbufs += [
        pltpu.VMEM((chunk,), jnp.int32),
        pltpu.VMEM((chunk,), jnp.int32),
        pltpu.VMEM((chunk,), jnp.float32),
        pltpu.VMEM((chunk,), jnp.float32),
    ]

  @functools.partial(
      pl.kernel,
      mesh=mesh,
      out_type=jax.ShapeDtypeStruct((_NC, n_pad), jnp.float32),
      compiler_params=pltpu.CompilerParams(needs_layout_passes=False),
      scratch_types=(
          [pltpu.VMEM((n,), jnp.float32)]
          + vbufs
          + [pltpu.VMEM_SHARED((n_pad,), jnp.float32)]
          + [pltpu.SemaphoreType.DMA] * (2 * _NBUF)
      ),
  )
  def spmv(ei_hbm, mv_hbm, d_hbm, out_hbm, d_t, *rest):
    bufs = [rest[4 * b:4 * b + 4] for b in range(_NBUF)]
    ad_s = rest[4 * _NBUF]
    lsem = rest[4 * _NBUF + 1:4 * _NBUF + 1 + _NBUF]
    ssem = rest[4 * _NBUF + 1 + _NBUF:]
    c = lax.axis_index("c")
    s = lax.axis_index("s")
    wid = c * _NS + s
    tbase = wid * ept

    d_copy = pltpu.async_copy(d_hbm, d_t, lsem[0])

    zv = bufs[0][3]

    def zbody(j, carry):
      zv[pl.ds(j * _LANES, _LANES)] = jnp.zeros((_LANES,), jnp.float32)
      return carry

    lax.fori_loop(0, chunk // _LANES, zbody, 0)
    for q in range(sl // chunk):
      pltpu.sync_copy(zv, ad_s.at[pl.ds(s * sl + q * chunk, chunk)])
    rem = sl % chunk
    if rem:
      pltpu.sync_copy(zv.at[pl.ds(0, rem)],
                      ad_s.at[pl.ds(s * sl + (sl // chunk) * chunk, rem)])
    d_copy.wait()
    plsc.subcore_barrier()

    def issue_loads(i):
      b = i % _NBUF
      base = tbase + i * chunk
      return [
          pltpu.async_copy(ei_hbm.at[pl.ds(base, chunk)], bufs[b][0], lsem[b]),
          pltpu.async_copy(ei_hbm.at[pl.ds(e + base, chunk)], bufs[b][1],
                           lsem[b]),
          pltpu.async_copy(mv_hbm.at[pl.ds(base, chunk)], bufs[b][2], lsem[b]),
      ]

    pending_loads = {0: issue_loads(0)}
    pending_scat = {}
    for i in range(nch):
      b = i % _NBUF
      j = i - (_NBUF - 1)
      if j in pending_scat:
        pending_scat.pop(j).wait()
      if i + 1 < nch:
        pending_loads[i + 1] = issue_loads(i + 1)
      for h in pending_loads.pop(i):
        h.wait()
      src_v, dst_v, mv_v, msg_v = bufs[b]

      def grp(jj, c2, src_v=src_v, mv_v=mv_v, msg_v=msg_v):
        o = jj * _LANES
        idx = src_v[pl.ds(o, _LANES)]
        vals = plsc.load_gather(d_t, [idx])
        msg_v[pl.ds(o, _LANES)] = vals * mv_v[pl.ds(o, _LANES)]
        return c2

      lax.fori_loop(0, chunk // _LANES, grp, 0)
      pending_scat[i] = pltpu.async_copy(
          msg_v, ad_s.at[dst_v], ssem[b], add=True)
    for i in sorted(pending_scat):
      pending_scat[i].wait()

    plsc.subcore_barrier()
    off = s * sl
    pltpu.sync_copy(ad_s.at[pl.ds(off, sl)], out_hbm.at[c, pl.ds(off, sl)])

  return spmv(ei_flat, matrix_values, d)


def _tc_reduce(partial3, resp, maskp):
  def body(p_ref, r_ref, m_ref, o_ref):
    ad = p_ref[0] + p_ref[1]
    o_ref[0, 0] = jnp.sum(jnp.abs(ad * m_ref[...] - r_ref[...]))

  return pl.pallas_call(
      body,
      out_shape=jax.ShapeDtypeStruct((1, 1), jnp.float32),
      out_specs=pl.BlockSpec(memory_space=pltpu.SMEM),
  )(partial3, resp, maskp)


def kernel(d, L_values, edge_index, matrix_values, mask, residual):
  n = d.shape[0]
  n_pad = 102400
  partial = _sc_spmv(edge_index.reshape(-1), matrix_values, d, n_pad, 2000)
  nr = n_pad // 128
  maskp = jnp.pad(mask.astype(jnp.float32), (0, n_pad - n)).reshape(nr, 128)
  resp = jnp.pad(residual, (0, n_pad - n)).reshape(nr, 128)
  tot = _tc_reduce(partial.reshape(_NC, nr, 128), resp, maskp)
  return tot[0, 0] / n


# speedup vs baseline: 325.5692x; 1.0110x over previous
import functools

import jax
import jax.numpy as jnp
from jax import lax
from jax.experimental import pallas as pl
from jax.experimental.pallas import tpu as pltpu
from jax.experimental.pallas import tpu_sc as plsc

_NC = 2
_NS = 16
_LANES = 16
_NBUF = 3


def _sc_spmv(ei_flat, matrix_values, d, n_pad, chunk):
  n = d.shape[0]
  e = matrix_values.shape[0]
  n_tiles = _NC * _NS
  ept = e // n_tiles
  nch = ept // chunk
  sl = n_pad // _NS
  assert ept % chunk == 0 and chunk % _LANES == 0 and sl % 8 == 0

  mesh = plsc.VectorSubcoreMesh(core_axis_name="c", subcore_axis_name="s")

  vbufs = []
  for _ in range(_NBUF):
    vbufs += [
        pltpu.VMEM((chunk,), jnp.int32),
        pltpu.VMEM((chunk,), jnp.int32),
        pltpu.VMEM((chunk,), jnp.float32),
        pltpu.VMEM((chunk,), jnp.float32),
    ]

  @functools.partial(
      pl.kernel,
      mesh=mesh,
      out_type=jax.ShapeDtypeStruct((_NC, n_pad), jnp.float32),
      compiler_params=pltpu.CompilerParams(needs_layout_passes=False),
      scratch_types=(
          [pltpu.VMEM((n,), jnp.float32)]
          + vbufs
          + [pltpu.VMEM_SHARED((n_pad,), jnp.float32)]
          + [pltpu.SemaphoreType.DMA] * (2 * _NBUF)
      ),
  )
  def spmv(ei_hbm, mv_hbm, d_hbm, out_hbm, d_t, *rest):
    bufs = [rest[4 * b:4 * b + 4] for b in range(_NBUF)]
    ad_s = rest[4 * _NBUF]
    lsem = rest[4 * _NBUF + 1:4 * _NBUF + 1 + _NBUF]
    ssem = rest[4 * _NBUF + 1 + _NBUF:]
    c = lax.axis_index("c")
    s = lax.axis_index("s")
    wid = c * _NS + s
    tbase = wid * ept

    d_copy = pltpu.async_copy(d_hbm, d_t, lsem[0])

    def issue_loads(i):
      b = i % _NBUF
      base = tbase + i * chunk
      return [
          pltpu.async_copy(ei_hbm.at[pl.ds(base, chunk)], bufs[b][0], lsem[b]),
          pltpu.async_copy(ei_hbm.at[pl.ds(e + base, chunk)], bufs[b][1],
                           lsem[b]),
          pltpu.async_copy(mv_hbm.at[pl.ds(base, chunk)], bufs[b][2], lsem[b]),
      ]

    pending_loads = {0: issue_loads(0)}

    zv = bufs[0][3]

    def zbody(j, carry):
      zv[pl.ds(j * _LANES, _LANES)] = jnp.zeros((_LANES,), jnp.float32)
      return carry

    lax.fori_loop(0, chunk // _LANES, zbody, 0)
    zero_copies = [
        pltpu.async_copy(zv, ad_s.at[pl.ds(s * sl + q * chunk, chunk)],
                         ssem[0])
        for q in range(sl // chunk)
    ]
    rem = sl % chunk
    if rem:
      zero_copies.append(pltpu.async_copy(
          zv.at[pl.ds(0, rem)],
          ad_s.at[pl.ds(s * sl + (sl // chunk) * chunk, rem)], ssem[0]))
    for h in zero_copies:
      h.wait()
    d_copy.wait()
    plsc.subcore_barrier()

    pending_scat = {}
    for i in range(nch):
      b = i % _NBUF
      j = i - (_NBUF - 1)
      if j in pending_scat:
        pending_scat.pop(j).wait()
      if i + 1 < nch:
        pending_loads[i + 1] = issue_loads(i + 1)
      for h in pending_loads.pop(i):
        h.wait()
      src_v, dst_v, mv_v, msg_v = bufs[b]

      def grp(jj, c2, src_v=src_v, mv_v=mv_v, msg_v=msg_v):
        o = jj * _LANES
        idx = src_v[pl.ds(o, _LANES)]
        vals = plsc.load_gather(d_t, [idx])
        msg_v[pl.ds(o, _LANES)] = vals * mv_v[pl.ds(o, _LANES)]
        return c2

      lax.fori_loop(0, chunk // _LANES, grp, 0)
      pending_scat[i] = pltpu.async_copy(
          msg_v, ad_s.at[dst_v], ssem[b], add=True)
    for i in sorted(pending_scat):
      pending_scat[i].wait()

    plsc.subcore_barrier()
    off = s * sl
    pltpu.sync_copy(ad_s.at[pl.ds(off, sl)], out_hbm.at[c, pl.ds(off, sl)])

  return spmv(ei_flat, matrix_values, d)


def _tc_reduce(partial3, resp, maskp):
  def body(p_ref, r_ref, m_ref, o_ref):
    ad = p_ref[0] + p_ref[1]
    o_ref[0, 0] = jnp.sum(jnp.abs(ad * m_ref[...] - r_ref[...]))

  return pl.pallas_call(
      body,
      out_shape=jax.ShapeDtypeStruct((1, 1), jnp.float32),
      out_specs=pl.BlockSpec(memory_space=pltpu.SMEM),
  )(partial3, resp, maskp)


def kernel(d, L_values, edge_index, matrix_values, mask, residual):
  n = d.shape[0]
  n_pad = 102400
  partial = _sc_spmv(edge_index.reshape(-1), matrix_values, d, n_pad, 2000)
  nr = n_pad // 128
  maskp = jnp.pad(mask.astype(jnp.float32), (0, n_pad - n)).reshape(nr, 128)
  resp = jnp.pad(residual, (0, n_pad - n)).reshape(nr, 128)
  tot = _tc_reduce(partial.reshape(_NC, nr, 128), resp, maskp)
  return tot[0, 0] / n


# speedup vs baseline: 332.3394x; 1.0208x over previous
import functools

import jax
import jax.numpy as jnp
from jax import lax
from jax.experimental import pallas as pl
from jax.experimental.pallas import tpu as pltpu
from jax.experimental.pallas import tpu_sc as plsc

_NC = 2
_NS = 16
_LANES = 16
_NBUF = 3


def _sc_spmv(ei_flat, matrix_values, d, n_pad, chunk):
  n = d.shape[0]
  e = matrix_values.shape[0]
  n_tiles = _NC * _NS
  ept = e // n_tiles
  nch = ept // chunk
  sl = n_pad // _NS
  assert ept % chunk == 0 and chunk % _LANES == 0 and sl % 8 == 0

  mesh = plsc.VectorSubcoreMesh(core_axis_name="c", subcore_axis_name="s")

  vbufs = []
  for _ in range(_NBUF):
    vbufs += [
        pltpu.VMEM((chunk,), jnp.int32),
        pltpu.VMEM((chunk,), jnp.int32),
        pltpu.VMEM((chunk,), jnp.float32),
        pltpu.VMEM((chunk,), jnp.float32),
    ]

  @functools.partial(
      pl.kernel,
      mesh=mesh,
      out_type=jax.ShapeDtypeStruct((_NC, n_pad), jnp.float32),
      compiler_params=pltpu.CompilerParams(needs_layout_passes=False),
      scratch_types=(
          [pltpu.VMEM((n,), jnp.float32)]
          + vbufs
          + [pltpu.VMEM_SHARED((n_pad,), jnp.float32)]
          + [pltpu.SemaphoreType.DMA] * (2 * _NBUF)
      ),
  )
  def spmv(ei_hbm, mv_hbm, d_hbm, out_hbm, d_t, *rest):
    bufs = [rest[4 * b:4 * b + 4] for b in range(_NBUF)]
    ad_s = rest[4 * _NBUF]
    lsem = rest[4 * _NBUF + 1:4 * _NBUF + 1 + _NBUF]
    ssem = rest[4 * _NBUF + 1 + _NBUF:]
    c = lax.axis_index("c")
    s = lax.axis_index("s")
    wid = c * _NS + s
    tbase = wid * ept

    d_copy = pltpu.async_copy(d_hbm, d_t, lsem[0])

    def issue_loads(i):
      b = i % _NBUF
      base = tbase + i * chunk
      return [
          pltpu.async_copy(ei_hbm.at[pl.ds(base, chunk)], bufs[b][0], lsem[b]),
          pltpu.async_copy(ei_hbm.at[pl.ds(e + base, chunk)], bufs[b][1],
                           lsem[b]),
          pltpu.async_copy(mv_hbm.at[pl.ds(base, chunk)], bufs[b][2], lsem[b]),
      ]

    pending_loads = {0: issue_loads(0)}

    zv = bufs[0][3]

    def zbody(j, carry):
      zv[pl.ds(j * _LANES, _LANES)] = jnp.zeros((_LANES,), jnp.float32)
      return carry

    lax.fori_loop(0, chunk // _LANES, zbody, 0)
    zero_copies = [
        pltpu.async_copy(zv, ad_s.at[pl.ds(s * sl + q * chunk, chunk)],
                         ssem[0])
        for q in range(sl // chunk)
    ]
    rem = sl % chunk
    if rem:
      zero_copies.append(pltpu.async_copy(
          zv.at[pl.ds(0, rem)],
          ad_s.at[pl.ds(s * sl + (sl // chunk) * chunk, rem)], ssem[0]))
    for h in zero_copies:
      h.wait()
    d_copy.wait()
    plsc.subcore_barrier()

    pending_scat = {}
    for i in range(nch):
      b = i % _NBUF
      j = i - (_NBUF - 1)
      if j in pending_scat:
        pending_scat.pop(j).wait()
      if i + 1 < nch:
        pending_loads[i + 1] = issue_loads(i + 1)
      for h in pending_loads.pop(i):
        h.wait()
      src_v, dst_v, mv_v, msg_v = bufs[b]

      @plsc.parallel_loop(0, chunk, step=_LANES, unroll=8)
      def _(o, src_v=src_v, mv_v=mv_v, msg_v=msg_v):
        idx = src_v[pl.ds(o, _LANES)]
        vals = plsc.load_gather(d_t, [idx])
        msg_v[pl.ds(o, _LANES)] = vals * mv_v[pl.ds(o, _LANES)]

      pending_scat[i] = pltpu.async_copy(
          msg_v, ad_s.at[dst_v], ssem[b], add=True)
    for i in sorted(pending_scat):
      pending_scat[i].wait()

    plsc.subcore_barrier()
    off = s * sl
    pltpu.sync_copy(ad_s.at[pl.ds(off, sl)], out_hbm.at[c, pl.ds(off, sl)])

  return spmv(ei_flat, matrix_values, d)


def _tc_reduce(partial3, resp, maskp):
  def body(p_ref, r_ref, m_ref, o_ref):
    ad = p_ref[0] + p_ref[1]
    o_ref[0, 0] = jnp.sum(jnp.abs(ad * m_ref[...] - r_ref[...]))

  return pl.pallas_call(
      body,
      out_shape=jax.ShapeDtypeStruct((1, 1), jnp.float32),
      out_specs=pl.BlockSpec(memory_space=pltpu.SMEM),
  )(partial3, resp, maskp)


def kernel(d, L_values, edge_index, matrix_values, mask, residual):
  n = d.shape[0]
  n_pad = 102400
  partial = _sc_spmv(edge_index.reshape(-1), matrix_values, d, n_pad, 2000)
  nr = n_pad // 128
  maskp = jnp.pad(mask.astype(jnp.float32), (0, n_pad - n)).reshape(nr, 128)
  resp = jnp.pad(residual, (0, n_pad - n)).reshape(nr, 128)
  tot = _tc_reduce(partial.reshape(_NC, nr, 128), resp, maskp)
  return tot[0, 0] / n


# speedup vs baseline: 346.2667x; 1.0419x over previous
import functools

import jax
import jax.numpy as jnp
from jax import lax
from jax.experimental import pallas as pl
from jax.experimental.pallas import tpu as pltpu
from jax.experimental.pallas import tpu_sc as plsc

_NC = 2
_NS = 16
_LANES = 16
_NBUF = 4


def _sc_spmv(ei_flat, matrix_values, d, n_pad, chunk):
  n = d.shape[0]
  e = matrix_values.shape[0]
  n_tiles = _NC * _NS
  ept = e // n_tiles
  nch = ept // chunk
  sl = n_pad // _NS
  assert ept % chunk == 0 and chunk % _LANES == 0 and sl % 8 == 0

  mesh = plsc.VectorSubcoreMesh(core_axis_name="c", subcore_axis_name="s")

  vbufs = []
  for _ in range(_NBUF):
    vbufs += [
        pltpu.VMEM((chunk,), jnp.int32),
        pltpu.VMEM((chunk,), jnp.int32),
        pltpu.VMEM((chunk,), jnp.float32),
    ]

  @functools.partial(
      pl.kernel,
      mesh=mesh,
      out_type=jax.ShapeDtypeStruct((_NC, n_pad), jnp.float32),
      compiler_params=pltpu.CompilerParams(needs_layout_passes=False),
      scratch_types=(
          [pltpu.VMEM((n,), jnp.float32)]
          + vbufs
          + [pltpu.VMEM_SHARED((n_pad,), jnp.float32)]
          + [pltpu.SemaphoreType.DMA] * (2 * _NBUF)
      ),
  )
  def spmv(ei_hbm, mv_hbm, d_hbm, out_hbm, d_t, *rest):
    bufs = [rest[3 * b:3 * b + 3] for b in range(_NBUF)]
    ad_s = rest[3 * _NBUF]
    lsem = rest[3 * _NBUF + 1:3 * _NBUF + 1 + _NBUF]
    ssem = rest[3 * _NBUF + 1 + _NBUF:]
    c = lax.axis_index("c")
    s = lax.axis_index("s")
    wid = c * _NS + s
    tbase = wid * ept

    d_copy = pltpu.async_copy(d_hbm, d_t, lsem[0])

    def issue_loads(i):
      b = i % _NBUF
      base = tbase + i * chunk
      return [
          pltpu.async_copy(ei_hbm.at[pl.ds(base, chunk)], bufs[b][0], lsem[b]),
          pltpu.async_copy(ei_hbm.at[pl.ds(e + base, chunk)], bufs[b][1],
                           lsem[b]),
          pltpu.async_copy(mv_hbm.at[pl.ds(base, chunk)], bufs[b][2], lsem[b]),
      ]

    pending_loads = {0: issue_loads(0), 1: issue_loads(1)}

    zv = bufs[_NBUF - 1][2]

    def zbody(j, carry):
      zv[pl.ds(j * _LANES, _LANES)] = jnp.zeros((_LANES,), jnp.float32)
      return carry

    lax.fori_loop(0, chunk // _LANES, zbody, 0)
    zero_copies = [
        pltpu.async_copy(zv, ad_s.at[pl.ds(s * sl + q * chunk, chunk)],
                         ssem[0])
        for q in range(sl // chunk)
    ]
    rem = sl % chunk
    if rem:
      zero_copies.append(pltpu.async_copy(
          zv.at[pl.ds(0, rem)],
          ad_s.at[pl.ds(s * sl + (sl // chunk) * chunk, rem)], ssem[0]))
    for h in zero_copies:
      h.wait()
    d_copy.wait()
    plsc.subcore_barrier()

    pending_scat = {}
    for i in range(nch):
      b = i % _NBUF
      j = i + 2 - _NBUF
      if j in pending_scat:
        pending_scat.pop(j).wait()
      if i + 2 < nch:
        pending_loads[i + 2] = issue_loads(i + 2)
      for h in pending_loads.pop(i):
        h.wait()
      src_v, dst_v, mv_v = bufs[b]

      @plsc.parallel_loop(0, chunk, step=_LANES, unroll=8)
      def _(o, src_v=src_v, mv_v=mv_v):
        idx = src_v[pl.ds(o, _LANES)]
        vals = plsc.load_gather(d_t, [idx])
        mv_v[pl.ds(o, _LANES)] = vals * mv_v[pl.ds(o, _LANES)]

      pending_scat[i] = pltpu.async_copy(
          mv_v, ad_s.at[dst_v], ssem[b], add=True)
    for i in sorted(pending_scat):
      pending_scat[i].wait()

    plsc.subcore_barrier()
    off = s * sl
    pltpu.sync_copy(ad_s.at[pl.ds(off, sl)], out_hbm.at[c, pl.ds(off, sl)])

  return spmv(ei_flat, matrix_values, d)


def _tc_reduce(partial3, resp, maskp):
  def body(p_ref, r_ref, m_ref, o_ref):
    ad = p_ref[0] + p_ref[1]
    o_ref[0, 0] = jnp.sum(jnp.abs(ad * m_ref[...] - r_ref[...]))

  return pl.pallas_call(
      body,
      out_shape=jax.ShapeDtypeStruct((1, 1), jnp.float32),
      out_specs=pl.BlockSpec(memory_space=pltpu.SMEM),
  )(partial3, resp, maskp)


def kernel(d, L_values, edge_index, matrix_values, mask, residual):
  n = d.shape[0]
  n_pad = 102400
  partial = _sc_spmv(edge_index.reshape(-1), matrix_values, d, n_pad, 2000)
  nr = n_pad // 128
  maskp = jnp.pad(mask.astype(jnp.float32), (0, n_pad - n)).reshape(nr, 128)
  resp = jnp.pad(residual, (0, n_pad - n)).reshape(nr, 128)
  tot = _tc_reduce(partial.reshape(_NC, nr, 128), resp, maskp)
  return tot[0, 0] / n


# speedup vs baseline: 346.9689x; 1.0020x over previous
import functools

import jax
import jax.numpy as jnp
from jax import lax
from jax.experimental import pallas as pl
from jax.experimental.pallas import tpu as pltpu
from jax.experimental.pallas import tpu_sc as plsc

_NC = 2
_NS = 16
_LANES = 16
_NBUF = 4


def _sc_spmv(ei_flat, matrix_values, d, n_pad, chunk):
  n = d.shape[0]
  e = matrix_values.shape[0]
  n_tiles = _NC * _NS
  ept = e // n_tiles
  nch = ept // chunk
  sl = n_pad // _NS
  assert ept % chunk == 0 and chunk % _LANES == 0 and sl % 8 == 0

  mesh = plsc.VectorSubcoreMesh(core_axis_name="c", subcore_axis_name="s")

  vbufs = []
  for _ in range(_NBUF):
    vbufs += [
        pltpu.VMEM((chunk,), jnp.int32),
        pltpu.VMEM((chunk,), jnp.int32),
        pltpu.VMEM((chunk,), jnp.float32),
    ]

  @functools.partial(
      pl.kernel,
      mesh=mesh,
      out_type=jax.ShapeDtypeStruct((_NC, n_pad), jnp.float32),
      compiler_params=pltpu.CompilerParams(needs_layout_passes=False),
      scratch_types=(
          [pltpu.VMEM((n,), jnp.float32)]
          + vbufs
          + [pltpu.VMEM_SHARED((n_pad,), jnp.float32)]
          + [pltpu.SemaphoreType.DMA] * (3 * _NBUF)
      ),
  )
  def spmv(ei_hbm, mv_hbm, d_hbm, out_hbm, d_t, *rest):
    bufs = [rest[3 * b:3 * b + 3] for b in range(_NBUF)]
    ad_s = rest[3 * _NBUF]
    lsem = rest[3 * _NBUF + 1:3 * _NBUF + 1 + _NBUF]
    dsem = rest[3 * _NBUF + 1 + _NBUF:3 * _NBUF + 1 + 2 * _NBUF]
    ssem = rest[3 * _NBUF + 1 + 2 * _NBUF:]
    c = lax.axis_index("c")
    s = lax.axis_index("s")
    wid = c * _NS + s
    tbase = wid * ept

    d_copy = pltpu.async_copy(d_hbm, d_t, lsem[0])

    def issue_loads(i):
      b = i % _NBUF
      base = tbase + i * chunk
      return (
          [
              pltpu.async_copy(ei_hbm.at[pl.ds(base, chunk)], bufs[b][0],
                               lsem[b]),
              pltpu.async_copy(mv_hbm.at[pl.ds(base, chunk)], bufs[b][2],
                               lsem[b]),
          ],
          pltpu.async_copy(ei_hbm.at[pl.ds(e + base, chunk)], bufs[b][1],
                           dsem[b]),
      )

    pending_loads = {0: issue_loads(0), 1: issue_loads(1)}

    zv = bufs[_NBUF - 1][2]

    def zbody(j, carry):
      zv[pl.ds(j * _LANES, _LANES)] = jnp.zeros((_LANES,), jnp.float32)
      return carry

    lax.fori_loop(0, chunk // _LANES, zbody, 0)
    zero_copies = [
        pltpu.async_copy(zv, ad_s.at[pl.ds(s * sl + q * chunk, chunk)],
                         ssem[0])
        for q in range(sl // chunk)
    ]
    rem = sl % chunk
    if rem:
      zero_copies.append(pltpu.async_copy(
          zv.at[pl.ds(0, rem)],
          ad_s.at[pl.ds(s * sl + (sl // chunk) * chunk, rem)], ssem[0]))
    for h in zero_copies:
      h.wait()
    plsc.subcore_barrier()
    d_copy.wait()

    pending_scat = {}
    for i in range(nch):
      b = i % _NBUF
      j = i + 2 - _NBUF
      if j in pending_scat:
        pending_scat.pop(j).wait()
      if i + 2 < nch:
        pending_loads[i + 2] = issue_loads(i + 2)
      gates, dst_copy = pending_loads.pop(i)
      for h in gates:
        h.wait()
      src_v, dst_v, mv_v = bufs[b]

      @plsc.parallel_loop(0, chunk, step=_LANES, unroll=8)
      def _(o, src_v=src_v, mv_v=mv_v):
        idx = src_v[pl.ds(o, _LANES)]
        vals = plsc.load_gather(d_t, [idx])
        mv_v[pl.ds(o, _LANES)] = vals * mv_v[pl.ds(o, _LANES)]

      dst_copy.wait()
      pending_scat[i] = pltpu.async_copy(
          mv_v, ad_s.at[dst_v], ssem[b], add=True)
    for i in sorted(pending_scat):
      pending_scat[i].wait()

    plsc.subcore_barrier()
    off = s * sl
    pltpu.sync_copy(ad_s.at[pl.ds(off, sl)], out_hbm.at[c, pl.ds(off, sl)])

  return spmv(ei_flat, matrix_values, d)


def _tc_reduce(partial3, resp, maskp):
  def body(p_ref, r_ref, m_ref, o_ref):
    ad = p_ref[0] + p_ref[1]
    o_ref[0, 0] = jnp.sum(jnp.abs(ad * m_ref[...] - r_ref[...]))

  return pl.pallas_call(
      body,
      out_shape=jax.ShapeDtypeStruct((1, 1), jnp.float32),
      out_specs=pl.BlockSpec(memory_space=pltpu.SMEM),
  )(partial3, resp, maskp)


def kernel(d, L_values, edge_index, matrix_values, mask, residual):
  n = d.shape[0]
  n_pad = 102400
  partial = _sc_spmv(edge_index.reshape(-1), matrix_values, d, n_pad, 2000)
  nr = n_pad // 128
  maskp = jnp.pad(mask.astype(jnp.float32), (0, n_pad - n)).reshape(nr, 128)
  resp = jnp.pad(residual, (0, n_pad - n)).reshape(nr, 128)
  tot = _tc_reduce(partial.reshape(_NC, nr, 128), resp, maskp)
  return tot[0, 0] / n
